# Initial kernel scaffold; baseline (speedup 1.0000x reference)
import jax
import jax.numpy as jnp
from jax.experimental import pallas as pl


def kernel(x, edge_index, edge_weight, W1, b1, W2, b2):
    raise NotImplementedError("write your pallas kernel here")


# speedup vs baseline: 14.6120x; 14.6120x over previous
import functools

import jax
import jax.numpy as jnp
from jax import lax
from jax.experimental import pallas as pl
from jax.experimental.pallas import tpu as pltpu
from jax.experimental.pallas import tpu_sc as plsc

N = 10000
E = 320000
D_IN = 128
D_HID = 16
D_OUT = 40

N_PAD = 10240
NW = 32
CHUNK = 128
CPW = -(-E // (NW * CHUNK))
E_PAD = NW * CHUNK * CPW
NPS = N_PAD // 16

_f32 = jnp.float32
_i32 = jnp.int32


def _mesh():
    return plsc.VectorSubcoreMesh(core_axis_name="c", subcore_axis_name="s")


_SC_PARAMS = pltpu.CompilerParams(needs_layout_passes=False)


def _zero_rows(rows_v, d):
    def body(g, carry):
        for t in range(d // 16):
            rows_v[g, pl.ds(t * 16, 16)] = jnp.zeros((16,), _f32)
        return carry
    lax.fori_loop(0, CHUNK, body, 0)


def _zero_acc(rows_v, acc, s):
    for b in range(NPS // CHUNK):
        pltpu.sync_copy(rows_v, acc.at[pl.ds(s * NPS + b * CHUNK, CHUNK)])


@functools.partial(
    pl.kernel,
    out_type=jax.ShapeDtypeStruct((2 * N_PAD, 16), _f32),
    mesh=_mesh(),
    compiler_params=_SC_PARAMS,
    scratch_types=[
        pltpu.VMEM_SHARED((N_PAD, 16), _f32),
        pltpu.VMEM((CHUNK,), _i32),
        pltpu.VMEM((CHUNK,), _f32),
        pltpu.VMEM((CHUNK, 16), _f32),
    ],
)
def _sc_degree(dst_hbm, ew_hbm, out_hbm, acc, dst_v, ew_v, rows_v):
    c = lax.axis_index("c")
    s = lax.axis_index("s")
    _zero_rows(rows_v, 16)
    _zero_acc(rows_v, acc, s)
    plsc.subcore_barrier()
    w = s * 2 + c

    def chunk(j, carry):
        r = w * CPW + j
        pltpu.sync_copy(dst_hbm.at[r], dst_v)
        pltpu.sync_copy(ew_hbm.at[r], ew_v)
        def fill(g, cc):
            m = plsc.load_gather(ew_v, [jnp.full((16,), g, _i32)])
            rows_v[g, :] = m
            return cc

        lax.fori_loop(0, CHUNK, fill, 0, unroll=4)
        pltpu.sync_copy(rows_v, acc.at[dst_v], add=True)
        return carry

    lax.fori_loop(0, CPW, chunk, 0)
    plsc.subcore_barrier()
    pltpu.sync_copy(acc.at[pl.ds(s * NPS, NPS)],
                    out_hbm.at[pl.ds(c * N_PAD + s * NPS, NPS)])


def _make_sc_agg(d):
    @functools.partial(
        pl.kernel,
        out_type=jax.ShapeDtypeStruct((2 * N_PAD, d), _f32),
        mesh=_mesh(),
        compiler_params=_SC_PARAMS,
        scratch_types=[
            pltpu.VMEM_SHARED((N_PAD, d), _f32),
            pltpu.VMEM_SHARED((N_PAD, d), _f32),
            pltpu.VMEM((CHUNK,), _i32),
            pltpu.VMEM((CHUNK,), _i32),
            pltpu.VMEM((CHUNK,), _f32),
            pltpu.VMEM((CHUNK, d), _f32),
            pltpu.SemaphoreType.DMA,
        ],
    )
    def sc_agg(xw_hbm, src_hbm, dst_hbm, ew_hbm, out_hbm,
               acc, tab_sh, src_v, dst_v, ew_v, rows_v, sem):
        c = lax.axis_index("c")
        s = lax.axis_index("s")
        pltpu.sync_copy(xw_hbm.at[pl.ds(s * NPS, NPS)],
                        tab_sh.at[pl.ds(s * NPS, NPS)])
        _zero_rows(rows_v, d)
        _zero_acc(rows_v, acc, s)
        plsc.subcore_barrier()
        w = s * 2 + c

        def chunk(j, carry):
            r = w * CPW + j
            pltpu.sync_copy(src_hbm.at[r], src_v)
            pltpu.sync_copy(dst_hbm.at[r], dst_v)
            pltpu.sync_copy(ew_hbm.at[r], ew_v)
            pltpu.async_copy(tab_sh.at[src_v], rows_v, sem).wait()

            def scale(g, cc):
                m = plsc.load_gather(ew_v, [jnp.full((16,), g, _i32)])
                for t in range(d // 16):
                    rows_v[g, pl.ds(t * 16, 16)] = (
                        rows_v[g, pl.ds(t * 16, 16)] * m)
                return cc

            lax.fori_loop(0, CHUNK, scale, 0, unroll=4)
            pltpu.sync_copy(rows_v, acc.at[dst_v], add=True)
            return carry

        lax.fori_loop(0, CPW, chunk, 0)
        plsc.subcore_barrier()
        pltpu.sync_copy(acc.at[pl.ds(s * NPS, NPS)],
                        out_hbm.at[pl.ds(c * N_PAD + s * NPS, NPS)])

    return sc_agg


_sc_agg16 = _make_sc_agg(16)
_sc_agg48 = _make_sc_agg(48)


def _tc1_body(deg_ref, x_ref, w_ref, dinv_ref, xw1_ref, xw1s_ref):
    deg = deg_ref[0:N_PAD, 0:1] + deg_ref[N_PAD:2 * N_PAD, 0:1] + 1.0
    dinv = lax.rsqrt(deg)
    xw = jnp.dot(x_ref[...], w_ref[...], preferred_element_type=_f32)
    dinv_ref[...] = dinv
    xw1_ref[...] = xw
    xw1s_ref[...] = xw * dinv


def _tc2_body(agg_ref, xw1_ref, dinv_ref, w_ref, b_ref, xw2_ref, xw2s_ref):
    dinv = dinv_ref[...]
    aggsum = agg_ref[0:N_PAD, :] + agg_ref[N_PAD:2 * N_PAD, :]
    h = dinv * aggsum + dinv * dinv * xw1_ref[...] + b_ref[...]
    xw2 = jnp.dot(h, w_ref[...], preferred_element_type=_f32)
    xw2_ref[...] = xw2
    xw2s_ref[...] = xw2 * dinv


def _tc3_body(agg_ref, xw2_ref, dinv_ref, b_ref, out_ref):
    dinv = dinv_ref[...]
    aggsum = agg_ref[0:N_PAD, :] + agg_ref[N_PAD:2 * N_PAD, :]
    out_ref[...] = dinv * aggsum + dinv * dinv * xw2_ref[...] + b_ref[...]


def kernel(x, edge_index, edge_weight, W1, b1, W2, b2):
    src = edge_index[0].astype(_i32)
    dst = edge_index[1].astype(_i32)
    ew = edge_weight.astype(_f32)

    npad = E_PAD - E
    pad_idx = (N + (jnp.arange(npad, dtype=_i32) % (N_PAD - N))).astype(_i32)
    src_p = jnp.concatenate([src, pad_idx]).reshape(E_PAD // CHUNK, CHUNK)
    dst_p = jnp.concatenate([dst, pad_idx]).reshape(E_PAD // CHUNK, CHUNK)
    ew_p = jnp.concatenate([ew, jnp.zeros((npad,), _f32)]).reshape(
        E_PAD // CHUNK, CHUNK)

    x_p = jnp.pad(x, ((0, N_PAD - N), (0, 0)))
    w2_p = jnp.pad(W2, ((0, 0), (0, 48 - D_OUT)))
    b1_2d = b1.reshape(1, D_HID)
    b2_2d = jnp.pad(b2, (0, 48 - D_OUT)).reshape(1, 48)

    deg_sc = _sc_degree(dst_p, ew_p)

    dinv, xw1, xw1s = pl.pallas_call(
        _tc1_body,
        out_shape=[
            jax.ShapeDtypeStruct((N_PAD, 1), _f32),
            jax.ShapeDtypeStruct((N_PAD, D_HID), _f32),
            jax.ShapeDtypeStruct((N_PAD, D_HID), _f32),
        ],
    )(deg_sc, x_p, W1)

    agg1 = _sc_agg16(xw1s, src_p, dst_p, ew_p)

    xw2, xw2s = pl.pallas_call(
        _tc2_body,
        out_shape=[
            jax.ShapeDtypeStruct((N_PAD, 48), _f32),
            jax.ShapeDtypeStruct((N_PAD, 48), _f32),
        ],
    )(agg1, xw1, dinv, w2_p, b1_2d)

    agg2 = _sc_agg48(xw2s, src_p, dst_p, ew_p)

    out = pl.pallas_call(
        _tc3_body,
        out_shape=jax.ShapeDtypeStruct((N_PAD, 48), _f32),
    )(agg2, xw2, dinv, b2_2d)

    return out[:N, :D_OUT]
